# baseline (speedup 1.0000x reference)
import functools

import jax
import jax.numpy as jnp
from jax import lax
from jax.experimental import pallas as pl
from jax.experimental.pallas import tpu as pltpu
from jax.experimental.pallas import tpu_sc as plsc

D_MODEL = 64
_SCALE = 8.0
_CHUNK = 128
_LANES = 16
_NR = 8
_NO = 4
_K = 4


@functools.lru_cache(maxsize=None)
def _build(n_pos: int, n_blocks: int):
    info = plsc.get_sparse_core_info()
    nw = info.num_cores * info.num_subcores
    assert n_blocks == nw
    assert n_pos % _NR == 0
    mesh = plsc.VectorSubcoreMesh(core_axis_name="c", subcore_axis_name="s")

    @functools.partial(
        pl.kernel,
        mesh=mesh,
        out_type=jax.ShapeDtypeStruct(
            (n_pos, D_MODEL // 8, nw, 8, _CHUNK), jnp.float32
        ),
        scratch_types=[
            pltpu.VMEM((n_pos, _CHUNK), jnp.int32),
            pltpu.VMEM((_NR, _CHUNK, D_MODEL), jnp.float32),
            pltpu.VMEM((_NO, D_MODEL // 8, 8, _CHUNK), jnp.float32),
        ]
        + [pltpu.SemaphoreType.DMA] * (_NR + _NO),
        compiler_params=pltpu.CompilerParams(
            use_tc_tiling_on_sc=False, needs_layout_passes=False
        ),
    )
    def emb_kernel(xt_hbm, table_hbm, out_hbm, idx_v, rows_v, tile_v, *sems):
        rsems, osems = sems[:_NR], sems[_NR:]
        wid = lax.axis_index("s") * info.num_cores + lax.axis_index("c")
        pltpu.sync_copy(xt_hbm.at[:, pl.ds(wid * _CHUNK, _CHUNK)], idx_v)

        iota = lax.iota(jnp.int32, _LANES)
        rowsel = [g * _LANES + iota for g in range(_CHUNK // _LANES)]

        def gather(c, b):
            pltpu.async_copy(table_hbm.at[idx_v.at[c]], rows_v.at[b], rsems[b])

        def wait_rows(b):
            pltpu.make_async_copy(
                table_hbm.at[pl.ds(0, _CHUNK)], rows_v.at[b], rsems[b]
            ).wait()

        def store(c, o):
            pltpu.async_copy(
                tile_v.at[o], out_hbm.at[c, :, wid], osems[o]
            )

        def wait_store(o):
            pltpu.make_async_copy(
                tile_v.at[o], out_hbm.at[0, :, wid], osems[o]
            ).wait()

        def transpose_scale(b, o):
            def fb_body(fb, carry):
                for fi in range(8):
                    col = jnp.full((_LANES,), fb * 8 + fi, jnp.int32)
                    for g in range(_CHUNK // _LANES):
                        vals = plsc.load_gather(rows_v.at[b], [rowsel[g], col])
                        tile_v[o, fb, fi, pl.ds(g * _LANES, _LANES)] = (
                            vals * _SCALE
                        )
                return carry

            lax.fori_loop(0, 8, fb_body, 0)

        for b in range(_K):
            gather(b, b)

        def step(s, carry):
            c0 = s * _NR
            for j in range(_NR):
                c = c0 + j
                b = j
                o = j % _NO
                wait_rows(b)

                @pl.when(c >= _NO)
                def _():
                    wait_store(o)

                transpose_scale(b, o)
                store(c, o)

                @pl.when(c + _K < n_pos)
                def _():
                    gather(c + _K, (j + _K) % _NR)

            return carry

        lax.fori_loop(0, n_pos // _NR, step, 0)
        for o in range(_NO):
            wait_store(o)

    return emb_kernel


def kernel(x, table):
    s0, s1 = x.shape
    info = plsc.get_sparse_core_info()
    nw = info.num_cores * info.num_subcores
    xt = x.astype(jnp.int32).T
    out5 = _build(s1, nw)(xt, table)
    out = out5.transpose(2, 4, 0, 1, 3).reshape(s0, s1, D_MODEL)
    return out


# speedup vs baseline: 1.7570x; 1.7570x over previous
import functools

import jax
import jax.numpy as jnp
from jax import lax
from jax.experimental import pallas as pl
from jax.experimental.pallas import tpu as pltpu
from jax.experimental.pallas import tpu_sc as plsc

D_MODEL = 64
_SCALE = 8.0
_CHUNK = 128
_LANES = 16
_NR = 8
_NO = 4
_K = 4
_PITCH = 129


@functools.lru_cache(maxsize=None)
def _build(n_pos: int, n_blocks: int):
    info = plsc.get_sparse_core_info()
    nw = info.num_cores * info.num_subcores
    assert n_blocks == nw and n_pos % 8 == 0
    n_pb = n_pos // 8
    mesh = plsc.VectorSubcoreMesh(core_axis_name="c", subcore_axis_name="s")

    @functools.partial(
        pl.kernel,
        mesh=mesh,
        out_type=jax.ShapeDtypeStruct(
            (n_pos, D_MODEL // 8, nw, 8, _CHUNK), jnp.float32
        ),
        scratch_types=[
            pltpu.VMEM((n_pb, 8, _CHUNK), jnp.int32),
            pltpu.VMEM((_NR, _CHUNK, D_MODEL), jnp.float32),
            pltpu.VMEM((_NO, D_MODEL // 8, 8, _PITCH), jnp.float32),
        ]
        + [pltpu.SemaphoreType.DMA] * (_NR + _NO),
        compiler_params=pltpu.CompilerParams(
            use_tc_tiling_on_sc=False, needs_layout_passes=False
        ),
    )
    def emb_kernel(xv_hbm, table_hbm, out_hbm, idx_v, rows_v, tile_v, *sems):
        rsems, osems = sems[:_NR], sems[_NR:]
        wid = lax.axis_index("s") * info.num_cores + lax.axis_index("c")
        pltpu.sync_copy(xv_hbm.at[:, wid], idx_v)

        iota = lax.iota(jnp.int32, _LANES)
        fbsel = [
            lax.shift_right_logical(g * _LANES + iota, 3)
            for g in range(D_MODEL // _LANES)
        ]
        fisel = [
            lax.bitwise_and(g * _LANES + iota, 7)
            for g in range(D_MODEL // _LANES)
        ]

        def gather(pb, pi, b):
            pltpu.async_copy(
                table_hbm.at[idx_v.at[pb, pi]], rows_v.at[b], rsems[b]
            )

        def wait_rows(b):
            pltpu.make_async_copy(
                table_hbm.at[pl.ds(0, _CHUNK)], rows_v.at[b], rsems[b]
            ).wait()

        def tile_out_src(o):
            return tile_v.at[o, :, :, pl.ds(0, _CHUNK)]

        def store(c, o):
            pltpu.async_copy(tile_out_src(o), out_hbm.at[c, :, wid], osems[o])

        def wait_store(o):
            pltpu.make_async_copy(
                tile_out_src(o), out_hbm.at[0, :, wid], osems[o]
            ).wait()

        def transpose_scale(b, o):
            def row_body(r, carry):
                bsel = jnp.full((_LANES,), r, jnp.int32)
                for g in range(D_MODEL // _LANES):
                    vals = rows_v[b, r, pl.ds(g * _LANES, _LANES)] * _SCALE
                    plsc.store_scatter(
                        tile_v.at[o], [fbsel[g], fisel[g], bsel], vals
                    )
                return carry

            lax.fori_loop(0, _CHUNK, row_body, 0, unroll=4)

        for j in range(_K):
            gather(0, j, j)

        def step(s, carry):
            for j in range(8):
                c = s * 8 + j
                b = j
                o = j % _NO
                wait_rows(b)

                @pl.when(c >= _NO)
                def _():
                    wait_store(o)

                transpose_scale(b, o)
                store(c, o)

                @pl.when(c + _K < n_pos)
                def _():
                    if j < 8 - _K:
                        gather(s, j + _K, (j + _K) % _NR)
                    else:
                        gather(s + 1, j + _K - 8, (j + _K) % _NR)

            return carry

        lax.fori_loop(0, n_pb, step, 0)
        for o in range(_NO):
            wait_store(o)

    return emb_kernel


def kernel(x, table):
    s0, s1 = x.shape
    info = plsc.get_sparse_core_info()
    nw = info.num_cores * info.num_subcores
    xv = (
        x.astype(jnp.int32)
        .reshape(nw, _CHUNK, s1 // 8, 8)
        .transpose(2, 0, 3, 1)
    )
    out5 = _build(s1, nw)(xv, table)
    out = out5.transpose(2, 4, 0, 1, 3).reshape(s0, s1, D_MODEL)
    return out


# speedup vs baseline: 2.6113x; 1.4862x over previous
import functools

import jax
import jax.numpy as jnp
from jax import lax
from jax.experimental import pallas as pl
from jax.experimental.pallas import tpu as pltpu
from jax.experimental.pallas import tpu_sc as plsc

D_MODEL = 64
_SCALE = 8.0
_CHUNK = 128
_LANES = 16
_NR = 8
_NO = 4
_K = 4
_PITCH = 129


@functools.lru_cache(maxsize=None)
def _build(n_pos: int, n_blocks: int):
    info = plsc.get_sparse_core_info()
    nw = info.num_cores * info.num_subcores
    assert n_blocks == nw and n_pos % 8 == 0
    n_pb = n_pos // 8
    mesh = plsc.VectorSubcoreMesh(core_axis_name="c", subcore_axis_name="s")

    @functools.partial(
        pl.kernel,
        mesh=mesh,
        out_type=jax.ShapeDtypeStruct(
            (n_pos, D_MODEL // 8, nw, 8, _CHUNK), jnp.float32
        ),
        scratch_types=[
            pltpu.VMEM((n_pb, 8, _CHUNK), jnp.int32),
            pltpu.VMEM((_NR, _CHUNK, D_MODEL), jnp.float32),
            pltpu.VMEM((_NO, D_MODEL // 8, 8, _PITCH), jnp.float32),
        ]
        + [pltpu.SemaphoreType.DMA] * (_NR + _NO),
        compiler_params=pltpu.CompilerParams(
            use_tc_tiling_on_sc=False, needs_layout_passes=False
        ),
    )
    def emb_kernel(xv_hbm, table_hbm, out_hbm, idx_v, rows_v, tile_v, *sems):
        rsems, osems = sems[:_NR], sems[_NR:]
        wid = lax.axis_index("s") * info.num_cores + lax.axis_index("c")
        pltpu.sync_copy(xv_hbm.at[:, wid], idx_v)

        iota = lax.iota(jnp.int32, _LANES)
        fbsel = [
            lax.shift_right_logical(g * _LANES + iota, 3)
            for g in range(D_MODEL // _LANES)
        ]
        fisel = [
            lax.bitwise_and(g * _LANES + iota, 7)
            for g in range(D_MODEL // _LANES)
        ]

        def gather(pb, pi, b):
            pltpu.async_copy(
                table_hbm.at[idx_v.at[pb, pi]], rows_v.at[b], rsems[b]
            )

        def wait_rows(b):
            pltpu.make_async_copy(
                table_hbm.at[pl.ds(0, _CHUNK)], rows_v.at[b], rsems[b]
            ).wait()

        def tile_out_src(o):
            return tile_v.at[o, :, :, pl.ds(0, _CHUNK)]

        def store(c, o):
            pltpu.async_copy(tile_out_src(o), out_hbm.at[c, :, wid], osems[o])

        def wait_store(o):
            pltpu.make_async_copy(
                tile_out_src(o), out_hbm.at[0, :, wid], osems[o]
            ).wait()

        def transpose_scale(b, o):
            @plsc.parallel_loop(0, _CHUNK, unroll=8)
            def row_body(r):
                bsel = jnp.full((_LANES,), r, jnp.int32)
                for g in range(D_MODEL // _LANES):
                    vals = rows_v[b, r, pl.ds(g * _LANES, _LANES)] * _SCALE
                    plsc.store_scatter(
                        tile_v.at[o], [fbsel[g], fisel[g], bsel], vals
                    )

        for j in range(_K):
            gather(0, j, j)

        def step(s, carry):
            for j in range(8):
                c = s * 8 + j
                b = j
                o = j % _NO
                wait_rows(b)

                @pl.when(c >= _NO)
                def _():
                    wait_store(o)

                transpose_scale(b, o)
                store(c, o)

                @pl.when(c + _K < n_pos)
                def _():
                    if j < 8 - _K:
                        gather(s, j + _K, (j + _K) % _NR)
                    else:
                        gather(s + 1, j + _K - 8, (j + _K) % _NR)

            return carry

        lax.fori_loop(0, n_pb, step, 0)
        for o in range(_NO):
            wait_store(o)

    return emb_kernel


def kernel(x, table):
    s0, s1 = x.shape
    info = plsc.get_sparse_core_info()
    nw = info.num_cores * info.num_subcores
    xv = (
        x.astype(jnp.int32)
        .reshape(nw, _CHUNK, s1 // 8, 8)
        .transpose(2, 0, 3, 1)
    )
    out5 = _build(s1, nw)(xv, table)
    out = out5.transpose(2, 4, 0, 1, 3).reshape(s0, s1, D_MODEL)
    return out
